# baseline (speedup 1.0000x reference)
import functools

import jax
import jax.numpy as jnp
from jax import lax
from jax.experimental import pallas as pl
from jax.experimental.pallas import tpu as pltpu
from jax.experimental.pallas import tpu_sc as plsc

N = 10000
E = 320000
D_IN = 128
D_H = 128
D_OUT = 64

NC = 2
NS = 16
K = 125
NBLK_DEG = E // NS // K
NBLK_AGG = E // NC // NS // K

_f32 = jnp.float32


def _make_deg_kernel():
    mesh = plsc.VectorSubcoreMesh(
        core_axis_name="c", subcore_axis_name="s", num_cores=NC, num_subcores=NS)

    @functools.partial(
        pl.kernel,
        out_type=(jax.ShapeDtypeStruct((N,), _f32),
                  jax.ShapeDtypeStruct((N,), _f32)),
        mesh=mesh,
        scratch_types=[
            pltpu.VMEM_SHARED((N,), _f32),
            pltpu.VMEM((NBLK_DEG, K), jnp.int32),
            pltpu.VMEM((K,), _f32),
        ],
    )
    def deg_kernel(edges_hbm, ones_hbm, zeros_hbm, odeg_hbm, ideg_hbm,
                   acc, idxl, onesl):
        c = lax.axis_index("c")
        s = lax.axis_index("s")
        @pl.when(s == 0)
        def _():
            pltpu.sync_copy(zeros_hbm, acc)
        pltpu.sync_copy(edges_hbm.at[c, s], idxl)
        pltpu.sync_copy(ones_hbm, onesl)
        plsc.subcore_barrier()

        def body(j, carry):
            pltpu.sync_copy(onesl, acc.at[idxl.at[j]], add=True)
            return carry
        lax.fori_loop(0, NBLK_DEG, body, 0)
        plsc.subcore_barrier()

        @pl.when((c == 0) & (s == 0))
        def _():
            pltpu.sync_copy(acc, odeg_hbm)
        @pl.when((c == 1) & (s == 0))
        def _():
            pltpu.sync_copy(acc, ideg_hbm)

    return deg_kernel


def _make_agg_kernel(d: int, tc_tiling: bool = True):
    mesh = plsc.VectorSubcoreMesh(
        core_axis_name="c", subcore_axis_name="s", num_cores=NC, num_subcores=NS)

    @functools.partial(
        pl.kernel,
        out_type=jax.ShapeDtypeStruct((NC, N, d), _f32),
        mesh=mesh,
        compiler_params=pltpu.CompilerParams(use_tc_tiling_on_sc=tc_tiling),
        scratch_types=[
            pltpu.VMEM_SHARED((N, d), _f32),
            pltpu.VMEM((NBLK_AGG // 2, K), jnp.int32),
            pltpu.VMEM((NBLK_AGG // 2, K), jnp.int32),
            pltpu.VMEM((K, d), _f32),
            pltpu.VMEM((K, d), _f32),
            pltpu.SemaphoreType.DMA,
            pltpu.SemaphoreType.DMA,
            pltpu.SemaphoreType.DMA,
            pltpu.SemaphoreType.DMA,
        ],
    )
    def agg_kernel(t_hbm, src_hbm, dst_hbm, zeros_hbm, out_hbm,
                   acc, srcl, dstl, rows0, rows1, gsem0, gsem1, ssem0, ssem1):
        c = lax.axis_index("c")
        s = lax.axis_index("s")
        rows_a = 624
        @pl.when(s < NS - 1)
        def _():
            pltpu.sync_copy(zeros_hbm.at[pl.ds(s * rows_a, rows_a)],
                            acc.at[pl.ds(s * rows_a, rows_a)])
        @pl.when(s == NS - 1)
        def _():
            last = N - (NS - 1) * rows_a
            pltpu.sync_copy(zeros_hbm.at[pl.ds((NS - 1) * rows_a, last)],
                            acc.at[pl.ds((NS - 1) * rows_a, last)])
        plsc.subcore_barrier()

        bufs = ((rows0, gsem0, ssem0), (rows1, gsem1, ssem1))
        nh = NBLK_AGG // 2

        for h in range(2):
            pltpu.sync_copy(src_hbm.at[c, s, pl.ds(h * nh, nh)], srcl)
            pltpu.sync_copy(dst_hbm.at[c, s, pl.ds(h * nh, nh)], dstl)
            pltpu.async_copy(t_hbm.at[srcl.at[0]], rows0, gsem0)

            def outer(jj, carry):
                for b in range(2):
                    j = 2 * jj + b
                    rows_b, gsem_b, ssem_b = bufs[b]
                    rows_n, gsem_n, ssem_n = bufs[1 - b]
                    @pl.when((j + 1 < nh) & (j >= 1))
                    def _():
                        pltpu.make_async_copy(
                            rows_n, acc.at[dstl.at[j - 1]], ssem_n).wait()
                    @pl.when(j + 1 < nh)
                    def _():
                        pltpu.async_copy(t_hbm.at[srcl.at[j + 1]], rows_n, gsem_n)
                    pltpu.make_async_copy(t_hbm.at[srcl.at[j]], rows_b, gsem_b).wait()
                    pltpu.async_copy(rows_b, acc.at[dstl.at[j]], ssem_b, add=True)
                return carry
            lax.fori_loop(0, nh // 2, outer, 0)
            pltpu.make_async_copy(rows0, acc.at[dstl.at[nh - 2]], ssem0).wait()
            pltpu.make_async_copy(rows1, acc.at[dstl.at[nh - 1]], ssem1).wait()
        plsc.subcore_barrier()

        rows_a = 624
        @pl.when(s < NS - 1)
        def _():
            pltpu.sync_copy(acc.at[pl.ds(s * rows_a, rows_a)],
                            out_hbm.at[c, pl.ds(s * rows_a, rows_a)])
        @pl.when(s == NS - 1)
        def _():
            last = N - (NS - 1) * rows_a
            pltpu.sync_copy(acc.at[pl.ds((NS - 1) * rows_a, last)],
                            out_hbm.at[c, pl.ds((NS - 1) * rows_a, last)])

    return agg_kernel


BN = 1000


def _norm(deg):
    return lax.rsqrt(jnp.where(deg > 0, deg, 1.0))


def _matmul_body(x_ref, w_ref, out_ref):
    out_ref[...] = jnp.dot(x_ref[...], w_ref[...], preferred_element_type=_f32)


def _scale_body(odeg_ref, x_ref, out_ref):
    out_ref[...] = x_ref[...] * _norm(odeg_ref[...])


def _mid_body(agg_ref, ideg_ref, odeg_ref, b_ref, w_ref, out_ref):
    a = agg_ref[0] + agg_ref[1]
    inorm = _norm(ideg_ref[...])
    h = jnp.maximum(a * inorm + b_ref[...], 0.0)
    onorm = _norm(odeg_ref[...])
    out_ref[...] = jnp.dot(h * onorm, w_ref[...], preferred_element_type=_f32)


def _final_body(agg_ref, ideg_ref, b_ref, out_ref):
    a = agg_ref[0] + agg_ref[1]
    inorm = _norm(ideg_ref[...])
    out_ref[...] = a * inorm + b_ref[...]


def _matmul_tc(x, w):
    d_out = w.shape[1]
    return pl.pallas_call(
        _matmul_body,
        grid=(N // BN,),
        in_specs=[
            pl.BlockSpec((BN, x.shape[1]), lambda i: (i, 0)),
            pl.BlockSpec(w.shape, lambda i: (0, 0)),
        ],
        out_specs=pl.BlockSpec((BN, d_out), lambda i: (i, 0)),
        out_shape=jax.ShapeDtypeStruct((N, d_out), _f32),
    )(x, w)


def _scale_tc(odeg2, x):
    d = x.shape[1]
    return pl.pallas_call(
        _scale_body,
        grid=(N // BN,),
        in_specs=[
            pl.BlockSpec((BN, 1), lambda i: (i, 0)),
            pl.BlockSpec((BN, d), lambda i: (i, 0)),
        ],
        out_specs=pl.BlockSpec((BN, d), lambda i: (i, 0)),
        out_shape=jax.ShapeDtypeStruct((N, d), _f32),
    )(odeg2, x)


def _mid_tc(agg, ideg2, odeg2, b, w):
    d_in = w.shape[0]
    d_out = w.shape[1]
    return pl.pallas_call(
        _mid_body,
        grid=(N // BN,),
        in_specs=[
            pl.BlockSpec((NC, BN, d_in), lambda i: (0, i, 0)),
            pl.BlockSpec((BN, 1), lambda i: (i, 0)),
            pl.BlockSpec((BN, 1), lambda i: (i, 0)),
            pl.BlockSpec((d_in,), lambda i: (0,)),
            pl.BlockSpec(w.shape, lambda i: (0, 0)),
        ],
        out_specs=pl.BlockSpec((BN, d_out), lambda i: (i, 0)),
        out_shape=jax.ShapeDtypeStruct((N, d_out), _f32),
    )(agg, ideg2, odeg2, b, w)


def _final_tc(agg, ideg2, b):
    d = agg.shape[2]
    return pl.pallas_call(
        _final_body,
        grid=(N // BN,),
        in_specs=[
            pl.BlockSpec((NC, BN, d), lambda i: (0, i, 0)),
            pl.BlockSpec((BN, 1), lambda i: (i, 0)),
            pl.BlockSpec((d,), lambda i: (0,)),
        ],
        out_specs=pl.BlockSpec((BN, d), lambda i: (i, 0)),
        out_shape=jax.ShapeDtypeStruct((N, d), _f32),
    )(agg, ideg2, b)


def kernel(features, edge_index, W0, b0, W1, b1, W2, b2):
    edge_index = edge_index.astype(jnp.int32)
    src_r = edge_index[0].reshape(NC, NS, NBLK_AGG, K)
    dst_r = edge_index[1].reshape(NC, NS, NBLK_AGG, K)
    edges_r = edge_index.reshape(2, NS, NBLK_DEG, K)

    zeros1 = jnp.zeros((N,), _f32)
    zeros128 = jnp.zeros((N, D_H), _f32)
    zeros64 = jnp.zeros((N, D_OUT), _f32)
    ones_k = jnp.ones((K,), _f32)

    deg = _make_deg_kernel()
    agg128 = _make_agg_kernel(D_H)
    agg64 = _make_agg_kernel(D_OUT, tc_tiling=False)

    odeg, ideg = deg(edges_r, ones_k, zeros1)
    odeg2 = odeg.reshape(N, 1)
    ideg2 = ideg.reshape(N, 1)

    t0u = _matmul_tc(features, W0)
    t0 = _scale_tc(odeg2, t0u)
    a0 = agg128(t0, src_r, dst_r, zeros128)
    t1 = _mid_tc(a0, ideg2, odeg2, b0, W1)
    a1 = agg128(t1, src_r, dst_r, zeros128)
    t2 = _mid_tc(a1, ideg2, odeg2, b1, W2)
    a2 = agg64(t2, src_r, dst_r, zeros64)
    return _final_tc(a2, ideg2, b2)


# speedup vs baseline: 1.0183x; 1.0183x over previous
import functools

import jax
import jax.numpy as jnp
from jax import lax
from jax.experimental import pallas as pl
from jax.experimental.pallas import tpu as pltpu
from jax.experimental.pallas import tpu_sc as plsc

N = 10000
E = 320000
D_IN = 128
D_H = 128
D_OUT = 64

NC = 2
NS = 16
K = 125
NBLK_DEG = E // NS // K
NBLK_AGG = E // NC // NS // K

_f32 = jnp.float32


def _make_deg_kernel():
    mesh = plsc.VectorSubcoreMesh(
        core_axis_name="c", subcore_axis_name="s", num_cores=NC, num_subcores=NS)

    @functools.partial(
        pl.kernel,
        out_type=(jax.ShapeDtypeStruct((N,), _f32),
                  jax.ShapeDtypeStruct((N,), _f32)),
        mesh=mesh,
        scratch_types=[
            pltpu.VMEM_SHARED((N,), _f32),
            pltpu.VMEM((NBLK_DEG, K), jnp.int32),
            pltpu.VMEM((K,), _f32),
        ],
    )
    def deg_kernel(edges_hbm, ones_hbm, zeros_hbm, odeg_hbm, ideg_hbm,
                   acc, idxl, onesl):
        c = lax.axis_index("c")
        s = lax.axis_index("s")
        @pl.when(s == 0)
        def _():
            pltpu.sync_copy(zeros_hbm, acc)
        pltpu.sync_copy(edges_hbm.at[c, s], idxl)
        pltpu.sync_copy(ones_hbm, onesl)
        plsc.subcore_barrier()

        def body(j, carry):
            pltpu.sync_copy(onesl, acc.at[idxl.at[j]], add=True)
            return carry
        lax.fori_loop(0, NBLK_DEG, body, 0)
        plsc.subcore_barrier()

        @pl.when((c == 0) & (s == 0))
        def _():
            pltpu.sync_copy(acc, odeg_hbm)
        @pl.when((c == 1) & (s == 0))
        def _():
            pltpu.sync_copy(acc, ideg_hbm)

    return deg_kernel


def _make_agg_kernel(d: int, tc_tiling: bool = True):
    mesh = plsc.VectorSubcoreMesh(
        core_axis_name="c", subcore_axis_name="s", num_cores=NC, num_subcores=NS)

    @functools.partial(
        pl.kernel,
        out_type=jax.ShapeDtypeStruct((NC, N, d), _f32),
        mesh=mesh,
        compiler_params=pltpu.CompilerParams(use_tc_tiling_on_sc=tc_tiling),
        scratch_types=[
            pltpu.VMEM_SHARED((N, d), _f32),
            pltpu.VMEM((NBLK_AGG // 2, K), jnp.int32),
            pltpu.VMEM((NBLK_AGG // 2, K), jnp.int32),
            pltpu.VMEM((K, d), _f32),
            pltpu.VMEM((K, d), _f32),
            pltpu.SemaphoreType.DMA,
            pltpu.SemaphoreType.DMA,
            pltpu.SemaphoreType.DMA,
            pltpu.SemaphoreType.DMA,
        ],
    )
    def agg_kernel(t_hbm, src_hbm, dst_hbm, zeros_hbm, out_hbm,
                   acc, srcl, dstl, rows0, rows1, gsem0, gsem1, ssem0, ssem1):
        c = lax.axis_index("c")
        s = lax.axis_index("s")
        rows_a = 624
        @pl.when(s < NS - 1)
        def _():
            pltpu.sync_copy(zeros_hbm.at[pl.ds(s * rows_a, rows_a)],
                            acc.at[pl.ds(s * rows_a, rows_a)])
        @pl.when(s == NS - 1)
        def _():
            last = N - (NS - 1) * rows_a
            pltpu.sync_copy(zeros_hbm.at[pl.ds((NS - 1) * rows_a, last)],
                            acc.at[pl.ds((NS - 1) * rows_a, last)])
        plsc.subcore_barrier()

        bufs = ((rows0, gsem0, ssem0), (rows1, gsem1, ssem1))
        nh = NBLK_AGG // 2

        for h in range(2):
            pltpu.sync_copy(src_hbm.at[c, s, pl.ds(h * nh, nh)], srcl)
            pltpu.sync_copy(dst_hbm.at[c, s, pl.ds(h * nh, nh)], dstl)
            pltpu.async_copy(t_hbm.at[srcl.at[0]], rows0, gsem0)

            def outer(jj, carry):
                for b in range(2):
                    j = 2 * jj + b
                    rows_b, gsem_b, ssem_b = bufs[b]
                    rows_n, gsem_n, ssem_n = bufs[1 - b]
                    @pl.when((j + 1 < nh) & (j >= 1))
                    def _():
                        pltpu.make_async_copy(
                            rows_n, acc.at[dstl.at[j - 1]], ssem_n).wait()
                    @pl.when(j + 1 < nh)
                    def _():
                        pltpu.async_copy(t_hbm.at[srcl.at[j + 1]], rows_n, gsem_n)
                    pltpu.make_async_copy(t_hbm.at[srcl.at[j]], rows_b, gsem_b).wait()
                    pltpu.async_copy(rows_b, acc.at[dstl.at[j]], ssem_b, add=True)
                return carry
            lax.fori_loop(0, nh // 2, outer, 0)
            pltpu.make_async_copy(rows0, acc.at[dstl.at[nh - 2]], ssem0).wait()
            pltpu.make_async_copy(rows1, acc.at[dstl.at[nh - 1]], ssem1).wait()
        plsc.subcore_barrier()

        rows_a = 624
        @pl.when(s < NS - 1)
        def _():
            pltpu.sync_copy(acc.at[pl.ds(s * rows_a, rows_a)],
                            out_hbm.at[c, pl.ds(s * rows_a, rows_a)])
        @pl.when(s == NS - 1)
        def _():
            last = N - (NS - 1) * rows_a
            pltpu.sync_copy(acc.at[pl.ds((NS - 1) * rows_a, last)],
                            out_hbm.at[c, pl.ds((NS - 1) * rows_a, last)])

    return agg_kernel


BN = 1000


def _norm(deg):
    return lax.rsqrt(jnp.where(deg > 0, deg, 1.0))


def _first_body(odeg_ref, x_ref, w_ref, out_ref):
    onorm = _norm(odeg_ref[...])
    out_ref[...] = jnp.dot(x_ref[...] * onorm, w_ref[...],
                           preferred_element_type=_f32)


def _mid_body(agg_ref, ideg_ref, odeg_ref, b_ref, w_ref, out_ref):
    a = agg_ref[0] + agg_ref[1]
    inorm = _norm(ideg_ref[...])
    h = jnp.maximum(a * inorm + b_ref[...], 0.0)
    onorm = _norm(odeg_ref[...])
    out_ref[...] = jnp.dot(h * onorm, w_ref[...], preferred_element_type=_f32)


def _final_body(agg_ref, ideg_ref, b_ref, out_ref):
    a = agg_ref[0] + agg_ref[1]
    inorm = _norm(ideg_ref[...])
    out_ref[...] = a * inorm + b_ref[...]


def _first_tc(odeg2, x, w):
    d_out = w.shape[1]
    return pl.pallas_call(
        _first_body,
        grid=(N // BN,),
        in_specs=[
            pl.BlockSpec((BN, 1), lambda i: (i, 0)),
            pl.BlockSpec((BN, x.shape[1]), lambda i: (i, 0)),
            pl.BlockSpec(w.shape, lambda i: (0, 0)),
        ],
        out_specs=pl.BlockSpec((BN, d_out), lambda i: (i, 0)),
        out_shape=jax.ShapeDtypeStruct((N, d_out), _f32),
    )(odeg2, x, w)


def _mid_tc(agg, ideg2, odeg2, b, w):
    d_in = w.shape[0]
    d_out = w.shape[1]
    return pl.pallas_call(
        _mid_body,
        grid=(N // BN,),
        in_specs=[
            pl.BlockSpec((NC, BN, d_in), lambda i: (0, i, 0)),
            pl.BlockSpec((BN, 1), lambda i: (i, 0)),
            pl.BlockSpec((BN, 1), lambda i: (i, 0)),
            pl.BlockSpec((d_in,), lambda i: (0,)),
            pl.BlockSpec(w.shape, lambda i: (0, 0)),
        ],
        out_specs=pl.BlockSpec((BN, d_out), lambda i: (i, 0)),
        out_shape=jax.ShapeDtypeStruct((N, d_out), _f32),
    )(agg, ideg2, odeg2, b, w)


def _final_tc(agg, ideg2, b):
    d = agg.shape[2]
    return pl.pallas_call(
        _final_body,
        grid=(N // BN,),
        in_specs=[
            pl.BlockSpec((NC, BN, d), lambda i: (0, i, 0)),
            pl.BlockSpec((BN, 1), lambda i: (i, 0)),
            pl.BlockSpec((d,), lambda i: (0,)),
        ],
        out_specs=pl.BlockSpec((BN, d), lambda i: (i, 0)),
        out_shape=jax.ShapeDtypeStruct((N, d), _f32),
    )(agg, ideg2, b)


def kernel(features, edge_index, W0, b0, W1, b1, W2, b2):
    edge_index = edge_index.astype(jnp.int32)
    src_r = edge_index[0].reshape(NC, NS, NBLK_AGG, K)
    dst_r = edge_index[1].reshape(NC, NS, NBLK_AGG, K)
    edges_r = edge_index.reshape(2, NS, NBLK_DEG, K)

    zeros1 = jnp.zeros((N,), _f32)
    zeros128 = jnp.zeros((N, D_H), _f32)
    zeros64 = jnp.zeros((N, D_OUT), _f32)
    ones_k = jnp.ones((K,), _f32)

    deg = _make_deg_kernel()
    agg128 = _make_agg_kernel(D_H, tc_tiling=False)
    agg64 = _make_agg_kernel(D_OUT, tc_tiling=False)

    odeg, ideg = deg(edges_r, ones_k, zeros1)
    odeg2 = odeg.reshape(N, 1)
    ideg2 = ideg.reshape(N, 1)

    t0 = _first_tc(odeg2, features, W0)
    a0 = agg128(t0, src_r, dst_r, zeros128)
    t1 = _mid_tc(a0, ideg2, odeg2, b0, W1)
    a1 = agg128(t1, src_r, dst_r, zeros128)
    t2 = _mid_tc(a1, ideg2, odeg2, b1, W2)
    a2 = agg64(t2, src_r, dst_r, zeros64)
    return _final_tc(a2, ideg2, b2)


# speedup vs baseline: 1.0185x; 1.0002x over previous
import functools

import jax
import jax.numpy as jnp
from jax import lax
from jax.experimental import pallas as pl
from jax.experimental.pallas import tpu as pltpu
from jax.experimental.pallas import tpu_sc as plsc

N = 10000
E = 320000
D_IN = 128
D_H = 128
D_OUT = 64

NC = 2
NS = 16
K = 125
NBLK_DEG = E // NS // K
NBLK_AGG = E // NC // NS // K

_f32 = jnp.float32


def _make_deg_kernel():
    mesh = plsc.VectorSubcoreMesh(
        core_axis_name="c", subcore_axis_name="s", num_cores=NC, num_subcores=NS)

    @functools.partial(
        pl.kernel,
        out_type=(jax.ShapeDtypeStruct((N,), _f32),
                  jax.ShapeDtypeStruct((N,), _f32)),
        mesh=mesh,
        scratch_types=[
            pltpu.VMEM_SHARED((N,), _f32),
            pltpu.VMEM((NBLK_DEG, K), jnp.int32),
            pltpu.VMEM((K,), _f32),
        ],
    )
    def deg_kernel(edges_hbm, ones_hbm, zeros_hbm, odeg_hbm, ideg_hbm,
                   acc, idxl, onesl):
        c = lax.axis_index("c")
        s = lax.axis_index("s")
        @pl.when(s == 0)
        def _():
            pltpu.sync_copy(zeros_hbm, acc)
        pltpu.sync_copy(edges_hbm.at[c, s], idxl)
        pltpu.sync_copy(ones_hbm, onesl)
        plsc.subcore_barrier()

        def body(j, carry):
            pltpu.sync_copy(onesl, acc.at[idxl.at[j]], add=True)
            return carry
        lax.fori_loop(0, NBLK_DEG, body, 0)
        plsc.subcore_barrier()

        @pl.when((c == 0) & (s == 0))
        def _():
            pltpu.sync_copy(acc, odeg_hbm)
        @pl.when((c == 1) & (s == 0))
        def _():
            pltpu.sync_copy(acc, ideg_hbm)

    return deg_kernel


def _make_agg_kernel(d: int, tc_tiling: bool = True):
    mesh = plsc.VectorSubcoreMesh(
        core_axis_name="c", subcore_axis_name="s", num_cores=NC, num_subcores=NS)

    @functools.partial(
        pl.kernel,
        out_type=jax.ShapeDtypeStruct((NC, N, d), _f32),
        mesh=mesh,
        compiler_params=pltpu.CompilerParams(use_tc_tiling_on_sc=tc_tiling),
        scratch_types=[
            pltpu.VMEM_SHARED((N, d), _f32),
            pltpu.VMEM((NBLK_AGG // 2, K), jnp.int32),
            pltpu.VMEM((NBLK_AGG // 2, K), jnp.int32),
            pltpu.VMEM((K, d), _f32),
            pltpu.VMEM((K, d), _f32),
            pltpu.SemaphoreType.DMA,
            pltpu.SemaphoreType.DMA,
            pltpu.SemaphoreType.DMA,
            pltpu.SemaphoreType.DMA,
        ],
    )
    def agg_kernel(t_hbm, src_hbm, dst_hbm, zeros_hbm, out_hbm,
                   acc, srcl, dstl, rows0, rows1, gsem0, gsem1, ssem0, ssem1):
        c = lax.axis_index("c")
        s = lax.axis_index("s")
        rows_a = 624
        @pl.when(s < NS - 1)
        def _():
            pltpu.sync_copy(zeros_hbm.at[pl.ds(s * rows_a, rows_a)],
                            acc.at[pl.ds(s * rows_a, rows_a)])
        @pl.when(s == NS - 1)
        def _():
            last = N - (NS - 1) * rows_a
            pltpu.sync_copy(zeros_hbm.at[pl.ds((NS - 1) * rows_a, last)],
                            acc.at[pl.ds((NS - 1) * rows_a, last)])
        plsc.subcore_barrier()

        bufs = ((rows0, gsem0, ssem0), (rows1, gsem1, ssem1))
        nh = NBLK_AGG // 2

        for h in range(2):
            pltpu.sync_copy(src_hbm.at[c, s, pl.ds(h * nh, nh)], srcl)
            pltpu.sync_copy(dst_hbm.at[c, s, pl.ds(h * nh, nh)], dstl)
            pltpu.async_copy(t_hbm.at[srcl.at[0]], rows0, gsem0)

            def outer(jj, carry):
                for b in range(2):
                    j = 2 * jj + b
                    rows_b, gsem_b, ssem_b = bufs[b]
                    rows_n, gsem_n, ssem_n = bufs[1 - b]
                    @pl.when((j + 1 < nh) & (j >= 1))
                    def _():
                        pltpu.make_async_copy(
                            rows_n, acc.at[dstl.at[j - 1]], ssem_n).wait()
                    @pl.when(j + 1 < nh)
                    def _():
                        pltpu.async_copy(t_hbm.at[srcl.at[j + 1, pl.ds(0, 64)]],
                                         rows_n.at[pl.ds(0, 64)], gsem_n)
                        pltpu.async_copy(t_hbm.at[srcl.at[j + 1, pl.ds(64, K - 64)]],
                                         rows_n.at[pl.ds(64, K - 64)], gsem_n)
                    pltpu.make_async_copy(t_hbm.at[srcl.at[j, pl.ds(0, 64)]],
                                          rows_b.at[pl.ds(0, 64)], gsem_b).wait()
                    pltpu.make_async_copy(t_hbm.at[srcl.at[j, pl.ds(64, K - 64)]],
                                          rows_b.at[pl.ds(64, K - 64)], gsem_b).wait()
                    pltpu.async_copy(rows_b, acc.at[dstl.at[j]], ssem_b, add=True)
                return carry
            lax.fori_loop(0, nh // 2, outer, 0)
            pltpu.make_async_copy(rows0, acc.at[dstl.at[nh - 2]], ssem0).wait()
            pltpu.make_async_copy(rows1, acc.at[dstl.at[nh - 1]], ssem1).wait()
        plsc.subcore_barrier()

        rows_a = 624
        @pl.when(s < NS - 1)
        def _():
            pltpu.sync_copy(acc.at[pl.ds(s * rows_a, rows_a)],
                            out_hbm.at[c, pl.ds(s * rows_a, rows_a)])
        @pl.when(s == NS - 1)
        def _():
            last = N - (NS - 1) * rows_a
            pltpu.sync_copy(acc.at[pl.ds((NS - 1) * rows_a, last)],
                            out_hbm.at[c, pl.ds((NS - 1) * rows_a, last)])

    return agg_kernel


BN = 1000


def _norm(deg):
    return lax.rsqrt(jnp.where(deg > 0, deg, 1.0))


def _first_body(odeg_ref, x_ref, w_ref, out_ref):
    onorm = _norm(odeg_ref[...])
    out_ref[...] = jnp.dot(x_ref[...] * onorm, w_ref[...],
                           preferred_element_type=_f32)


def _mid_body(agg_ref, ideg_ref, odeg_ref, b_ref, w_ref, out_ref):
    a = agg_ref[0] + agg_ref[1]
    inorm = _norm(ideg_ref[...])
    h = jnp.maximum(a * inorm + b_ref[...], 0.0)
    onorm = _norm(odeg_ref[...])
    out_ref[...] = jnp.dot(h * onorm, w_ref[...], preferred_element_type=_f32)


def _final_body(agg_ref, ideg_ref, b_ref, out_ref):
    a = agg_ref[0] + agg_ref[1]
    inorm = _norm(ideg_ref[...])
    out_ref[...] = a * inorm + b_ref[...]


def _first_tc(odeg2, x, w):
    d_out = w.shape[1]
    return pl.pallas_call(
        _first_body,
        grid=(N // BN,),
        in_specs=[
            pl.BlockSpec((BN, 1), lambda i: (i, 0)),
            pl.BlockSpec((BN, x.shape[1]), lambda i: (i, 0)),
            pl.BlockSpec(w.shape, lambda i: (0, 0)),
        ],
        out_specs=pl.BlockSpec((BN, d_out), lambda i: (i, 0)),
        out_shape=jax.ShapeDtypeStruct((N, d_out), _f32),
    )(odeg2, x, w)


def _mid_tc(agg, ideg2, odeg2, b, w):
    d_in = w.shape[0]
    d_out = w.shape[1]
    return pl.pallas_call(
        _mid_body,
        grid=(N // BN,),
        in_specs=[
            pl.BlockSpec((NC, BN, d_in), lambda i: (0, i, 0)),
            pl.BlockSpec((BN, 1), lambda i: (i, 0)),
            pl.BlockSpec((BN, 1), lambda i: (i, 0)),
            pl.BlockSpec((d_in,), lambda i: (0,)),
            pl.BlockSpec(w.shape, lambda i: (0, 0)),
        ],
        out_specs=pl.BlockSpec((BN, d_out), lambda i: (i, 0)),
        out_shape=jax.ShapeDtypeStruct((N, d_out), _f32),
    )(agg, ideg2, odeg2, b, w)


def _final_tc(agg, ideg2, b):
    d = agg.shape[2]
    return pl.pallas_call(
        _final_body,
        grid=(N // BN,),
        in_specs=[
            pl.BlockSpec((NC, BN, d), lambda i: (0, i, 0)),
            pl.BlockSpec((BN, 1), lambda i: (i, 0)),
            pl.BlockSpec((d,), lambda i: (0,)),
        ],
        out_specs=pl.BlockSpec((BN, d), lambda i: (i, 0)),
        out_shape=jax.ShapeDtypeStruct((N, d), _f32),
    )(agg, ideg2, b)


def kernel(features, edge_index, W0, b0, W1, b1, W2, b2):
    edge_index = edge_index.astype(jnp.int32)
    src_r = edge_index[0].reshape(NC, NS, NBLK_AGG, K)
    dst_r = edge_index[1].reshape(NC, NS, NBLK_AGG, K)
    edges_r = edge_index.reshape(2, NS, NBLK_DEG, K)

    zeros1 = jnp.zeros((N,), _f32)
    zeros128 = jnp.zeros((N, D_H), _f32)
    zeros64 = jnp.zeros((N, D_OUT), _f32)
    ones_k = jnp.ones((K,), _f32)

    deg = _make_deg_kernel()
    agg128 = _make_agg_kernel(D_H, tc_tiling=False)
    agg64 = _make_agg_kernel(D_OUT, tc_tiling=False)

    odeg, ideg = deg(edges_r, ones_k, zeros1)
    odeg2 = odeg.reshape(N, 1)
    ideg2 = ideg.reshape(N, 1)

    t0 = _first_tc(odeg2, features, W0)
    a0 = agg128(t0, src_r, dst_r, zeros128)
    t1 = _mid_tc(a0, ideg2, odeg2, b0, W1)
    a1 = agg128(t1, src_r, dst_r, zeros128)
    t2 = _mid_tc(a1, ideg2, odeg2, b1, W2)
    a2 = agg64(t2, src_r, dst_r, zeros64)
    return _final_tc(a2, ideg2, b2)
